# Initial kernel scaffold; baseline (speedup 1.0000x reference)
import jax
import jax.numpy as jnp
from jax.experimental import pallas as pl


def kernel(coords, edge_features, timesteps, edge_index, params):
    raise NotImplementedError("write your pallas kernel here")


# speedup vs baseline: 6.1697x; 6.1697x over previous
import math

import jax
import jax.numpy as jnp
from jax.experimental import pallas as pl

N = 200
ND = 64
ED = 128
H = 128
NB = 8
GB = N // NB
R = NB * N


def _silu(v):
    return v * jax.nn.sigmoid(v)


def _ln(v, g, b):
    m = jnp.mean(v, -1, keepdims=True)
    var = jnp.mean((v - m) * (v - m), -1, keepdims=True)
    return (v - m) / jnp.sqrt(var + 1e-5) * g + b


def _dot(a, b):
    return jax.lax.dot_general(a, b, (((1,), (0,)), ((), ())),
                               preferred_element_type=jnp.float32)


def _dot_hi(a, b):
    return jax.lax.dot_general(a, b, (((1,), (0,)), ((), ())),
                               preferred_element_type=jnp.float32,
                               precision=jax.lax.Precision.HIGHEST)


def _layer_kernel(h_ref, x_ref, x4_ref, e_ref, tv_ref,
                  Whr, Whc, wd, Wme, bm1, gml, bml, Wm2, bm2, Wm3, bm3,
                  Wc1, bc1, Wc2,
                  Wn1a, Wn1b, bn1, gnl, bnl, Wn2, bn2, gnn, bnn,
                  We1a, We1b, be1, gel, bel, We2, be2, gen, ben,
                  h_out, x_out, e_out):
    i0 = pl.program_id(1) * NB
    hf = h_ref[0]
    et = e_ref[0] + tv_ref[0]

    hcW = _dot(hf, Whc[...])
    hcF = jnp.broadcast_to(hcW[None], (NB, N, H)).reshape(R, H)
    hb = h_ref[0, pl.ds(i0, NB), :]
    hrW = _dot(hb, Whr[...])
    hrF = jnp.broadcast_to(hrW[:, None, :], (NB, N, H)).reshape(R, H)

    xj0 = x_ref[0, :, 0:1].reshape(1, N, 1)
    xj1 = x_ref[0, :, 1:2].reshape(1, N, 1)
    xi0 = x4_ref[0, pl.ds(i0, NB), :, 0:1]
    xi1 = x4_ref[0, pl.ds(i0, NB), :, 1:2]
    dxF = (xj0 - xi0).reshape(R, 1)
    dyF = (xj1 - xi1).reshape(R, 1)
    distF = jnp.sqrt(dxF * dxF + dyF * dyF)

    m = et @ Wme[...] + hcF + hrF + distF * wd[...] + bm1[...]
    m = _ln(_silu(m), gml[...], bml[...])
    m = _silu(m @ Wm2[...] + bm2[...])
    msg = m @ Wm3[...] + bm3[...]

    ii = jax.lax.broadcasted_iota(jnp.int32, (NB, R), 0)
    rr = jax.lax.broadcasted_iota(jnp.int32, (NB, R), 1)
    seg = (rr >= ii * N) & (rr < (ii + 1) * N)
    ssum = jnp.where(seg, 1.0, 0.0)
    smask = jnp.where(seg & (rr != ii * (N + 1) + i0), 1.0, 0.0)

    cw = _silu(msg @ Wc1[...] + bc1[...]) @ Wc2[...]
    wF = cw / (distF + 1e-8)
    contrib = jnp.concatenate([wF * dxF, wF * dyF], axis=1)
    upd = _dot_hi(ssum, contrib)
    x_out[0] = x_ref[0, pl.ds(i0, NB), :] + upd

    hagg = _dot_hi(smask, msg)
    nh = _dot(hb, Wn1a[...]) + _dot(hagg, Wn1b[...]) + bn1[...]
    nh = _ln(_silu(nh), gnl[...], bnl[...])
    nh = _dot(nh, Wn2[...]) + bn2[...]
    h_out[0] = _ln(hb + nh, gnn[...], bnn[...])

    ne = et @ We1a[...] + msg @ We1b[...] + be1[...]
    ne = _ln(_silu(ne), gel[...], bel[...])
    ne = ne @ We2[...] + be2[...]
    e_out[0] = _ln(et + ne, gen[...], ben[...])


def _head_kernel(e_ref, gl1, bl1, Wo1, bo1, gl2, bl2, Wo2, bo2, Wo3, bo3,
                 o_out):
    o = _ln(e_ref[0], gl1[...], bl1[...])
    o = _ln(_silu(o @ Wo1[...] + bo1[...]), gl2[...], bl2[...])
    o = _silu(_dot(o, Wo2[...]) + bo2[...])
    o_out[0] = _dot(o, Wo3[...]) + bo3[...]


def _full(shape):
    nd = len(shape)
    return pl.BlockSpec(shape, lambda b, nb: (0,) * nd)


def _row2(v):
    return v.reshape(1, -1)


def _layer_weights(p):
    w1 = p["msg1"]["w"]
    return [
        w1[0:ND], w1[ND:2 * ND], w1[2 * ND:2 * ND + 1], w1[2 * ND + 1:],
        _row2(p["msg1"]["b"]), _row2(p["msg_ln"]["g"]), _row2(p["msg_ln"]["b"]),
        p["msg2"]["w"], _row2(p["msg2"]["b"]),
        p["msg3"]["w"], _row2(p["msg3"]["b"]),
        p["coord1"]["w"], _row2(p["coord1"]["b"]), p["coord2"]["w"],
        p["node1"]["w"][0:ND], p["node1"]["w"][ND:], _row2(p["node1"]["b"]),
        _row2(p["node_ln"]["g"]), _row2(p["node_ln"]["b"]),
        p["node2"]["w"], _row2(p["node2"]["b"]),
        _row2(p["node_norm"]["g"]), _row2(p["node_norm"]["b"]),
        p["edge1"]["w"][0:ED], p["edge1"]["w"][ED:], _row2(p["edge1"]["b"]),
        _row2(p["edge_ln"]["g"]), _row2(p["edge_ln"]["b"]),
        p["edge2"]["w"], _row2(p["edge2"]["b"]),
        _row2(p["edge_norm"]["g"]), _row2(p["edge_norm"]["b"]),
    ]


def _run_layer(h, x, e, tv, lw):
    bsz = h.shape[0]
    x4 = x.reshape(bsz, N, 1, 2)
    specs = [
        pl.BlockSpec((1, N, ND), lambda b, nb: (b, 0, 0)),
        pl.BlockSpec((1, N, 2), lambda b, nb: (b, 0, 0)),
        pl.BlockSpec((1, N, 1, 2), lambda b, nb: (b, 0, 0, 0)),
        pl.BlockSpec((1, R, ED), lambda b, nb: (b, nb, 0)),
        pl.BlockSpec((1, 1, ED), lambda b, nb: (b, 0, 0)),
    ] + [_full(w.shape) for w in lw]
    return pl.pallas_call(
        _layer_kernel,
        grid=(bsz, GB),
        in_specs=specs,
        out_specs=[
            pl.BlockSpec((1, NB, ND), lambda b, nb: (b, nb, 0)),
            pl.BlockSpec((1, NB, 2), lambda b, nb: (b, nb, 0)),
            pl.BlockSpec((1, R, ED), lambda b, nb: (b, nb, 0)),
        ],
        out_shape=[
            jax.ShapeDtypeStruct((bsz, N, ND), jnp.float32),
            jax.ShapeDtypeStruct((bsz, N, 2), jnp.float32),
            jax.ShapeDtypeStruct((bsz, N * N, ED), jnp.float32),
        ],
    )(h, x, x4, e, tv, *lw)


def _run_head(e, params):
    bsz = e.shape[0]
    hw = [
        _row2(params["out_ln1"]["g"]), _row2(params["out_ln1"]["b"]),
        params["out1"]["w"], _row2(params["out1"]["b"]),
        _row2(params["out_ln2"]["g"]), _row2(params["out_ln2"]["b"]),
        params["out2"]["w"], _row2(params["out2"]["b"]),
        params["out3"]["w"], _row2(params["out3"]["b"]),
    ]
    return pl.pallas_call(
        _head_kernel,
        grid=(bsz, GB),
        in_specs=[pl.BlockSpec((1, R, ED), lambda b, nb: (b, nb, 0))]
        + [_full(w.shape) for w in hw],
        out_specs=pl.BlockSpec((1, R, 2), lambda b, nb: (b, nb, 0)),
        out_shape=jax.ShapeDtypeStruct((bsz, N * N, 2), jnp.float32),
    )(e, *hw)


def _time_embedding(t, dim, max_period=10000):
    half = dim // 2
    freqs = jnp.exp(-math.log(max_period)
                    * jnp.arange(half, dtype=jnp.float32) / half)
    args = t[:, None].astype(jnp.float32) * freqs[None]
    return jnp.concatenate([jnp.cos(args), jnp.sin(args)], -1)


def kernel(coords, edge_features, timesteps, edge_index, params):
    bsz = coords.shape[0]
    row, col = edge_index[0], edge_index[1]
    idx = row * N + col

    h = coords @ params["node_embed"]["w"] + params["node_embed"]["b"]
    x = coords
    efd = jnp.zeros((bsz, N * N), jnp.float32).at[:, idx].set(edge_features)
    e = efd[..., None] * params["edge_embed"]["w"][0] + params["edge_embed"]["b"]

    t = _time_embedding(timesteps, H)
    t = _silu(t @ params["time1"]["w"] + params["time1"]["b"])
    t = t @ params["time2"]["w"] + params["time2"]["b"]

    for lp in params["layers"]:
        tv = (t @ lp["time"]["w"] + lp["time"]["b"]).reshape(bsz, 1, ED)
        h, x, e = _run_layer(h, x, e, tv, _layer_weights(lp))

    o = _run_head(e, params)
    return o[:, idx, :]
